# baseline (speedup 1.0000x reference)
import functools

import jax
import jax.numpy as jnp
from jax import lax
from jax.experimental import pallas as pl
from jax.experimental.pallas import tpu as pltpu
from jax.experimental.pallas import tpu_sc as plsc

MAX_LEN = 2048
NUM_HEADS = 16
PAD_W = 2 * MAX_LEN
NSHIFT = 8
GROUP = 8
GSEM = 4

_info = plsc.get_sparse_core_info()
_NC, _NS = _info.num_cores, _info.num_subcores
_NW = _NC * _NS
_ROWS_PER = (NUM_HEADS * MAX_LEN) // _NW
_WPH = MAX_LEN // _ROWS_PER


def _make_sc_kernel():
    mesh = plsc.VectorSubcoreMesh(core_axis_name="c", subcore_axis_name="s")

    @functools.partial(
        pl.kernel,
        mesh=mesh,
        out_type=jax.ShapeDtypeStruct((NUM_HEADS * MAX_LEN * MAX_LEN,), jnp.float32),
        scratch_types=[pltpu.VMEM((PAD_W,), jnp.float32)] * NSHIFT
        + [pltpu.VMEM((GROUP * MAX_LEN,), jnp.float32)]
        + [pltpu.SemaphoreType.DMA] * GSEM,
    )
    def sc_bias(tt8_hbm, out_hbm, *scratch):
        vs = scratch[:NSHIFT]
        drain_v = scratch[NSHIFT]
        sems = scratch[NSHIFT + 1 :]
        wid = lax.axis_index("s") * _NC + lax.axis_index("c")
        h = wid // _WPH
        i0 = (wid % _WPH) * _ROWS_PER

        for s in range(NSHIFT):
            pltpu.sync_copy(tt8_hbm.at[pl.ds((h * NSHIFT + s) * PAD_W, PAD_W)], vs[s])

        kmax = _ROWS_PER // NSHIFT
        ngrp = kmax // GROUP

        def group_wait(b):
            pltpu.make_async_copy(
                drain_v, out_hbm.at[pl.ds(0, GROUP * MAX_LEN)], sems[b]
            ).wait()

        def blk(g, carry):
            for b in range(GSEM):
                @pl.when(g > 0)
                def _wait():
                    group_wait(b)

                k = g * GSEM + b
                for r in range(NSHIFT):
                    s_r = (MAX_LEN - 1 - r) % NSHIFT
                    base = MAX_LEN - 1 - s_r - r - i0
                    i = i0 + NSHIFT * k + r
                    off = pl.multiple_of(base - NSHIFT * k, NSHIFT)
                    half = MAX_LEN // 2
                    for p in range(2):
                        src = vs[s_r].at[pl.ds(off + p * half, half)]
                        dst = out_hbm.at[
                            pl.ds((h * MAX_LEN + i) * MAX_LEN + p * half, half)
                        ]
                        pltpu.make_async_copy(src, dst, sems[b]).start()
            return carry

        lax.fori_loop(0, (_ROWS_PER // NSHIFT) // GSEM, blk, 0)

        for b in range(GSEM):
            group_wait(b)

    return sc_bias


_sc_bias = _make_sc_kernel()


@jax.jit
def kernel(T, table):
    del T
    ttp = jnp.pad(jnp.transpose(table), ((0, 0), (0, NSHIFT + 1)))
    tt8 = jnp.stack(
        [ttp[:, s : s + PAD_W] for s in range(NSHIFT)], axis=1
    )
    out = _sc_bias(tt8.reshape(-1))
    return out.reshape(NUM_HEADS, MAX_LEN, MAX_LEN)


# speedup vs baseline: 1.8044x; 1.8044x over previous
import functools

import jax
import jax.numpy as jnp
from jax import lax
from jax.experimental import pallas as pl
from jax.experimental.pallas import tpu as pltpu
from jax.experimental.pallas import tpu_sc as plsc

MAX_LEN = 2048
NUM_HEADS = 16
NU = 16
LM = 4224
LPAD = 136
WIN = 2944
NSC = 4

_info = plsc.get_sparse_core_info()
_NC, _NS = _info.num_cores, _info.num_subcores
_NW = _NC * _NS
_ROWS_PER = (NUM_HEADS * MAX_LEN) // _NW
_WPH = MAX_LEN // _ROWS_PER


def _make_sc_kernel():
    mesh = plsc.VectorSubcoreMesh(core_axis_name="c", subcore_axis_name="s")

    @functools.partial(
        pl.kernel,
        mesh=mesh,
        out_type=jax.ShapeDtypeStruct((NUM_HEADS, MAX_LEN, MAX_LEN), jnp.float32),
        scratch_types=[pltpu.VMEM((8, WIN), jnp.float32)] * 2
        + [pltpu.SemaphoreType.DMA] * (2 * NSC + 2),
    )
    def sc_bias(l3_hbm, out_hbm, *scratch):
        win = scratch[:2]
        sems = scratch[2 : 2 + 2 * NSC]
        ssem = scratch[2 + 2 * NSC :]
        wid = lax.axis_index("s") * _NC + lax.axis_index("c")
        h = wid // _WPH
        p = wid % _WPH

        mbase = pl.multiple_of(1152 - 1024 * p, 128)

        def stage(u, g):
            pltpu.make_async_copy(
                l3_hbm.at[u * NUM_HEADS + h, :, pl.ds(mbase, WIN)], win[g], ssem[g]
            ).start()

        def stage_wait(u, g):
            pltpu.make_async_copy(
                l3_hbm.at[u * NUM_HEADS + h, :, pl.ds(mbase, WIN)], win[g], ssem[g]
            ).wait()

        def scatter(u, g, n8):
            row0 = pl.multiple_of(8 * (u + 128 * p + 16 * n8), 8)
            return pltpu.make_async_copy(
                win[g].at[:, pl.ds(128 * (7 - n8), MAX_LEN)],
                out_hbm.at[h, pl.ds(row0, 8), :],
                sems[NSC * g + (n8 % NSC)],
            )

        stage(0, 0)
        for u in range(NU):
            g = u % 2
            stage_wait(u, g)
            for n8 in range(8):
                if n8 >= NSC:
                    scatter(u, g, n8 - NSC).wait()
                scatter(u, g, n8).start()
            if u + 1 < NU:
                if u >= 1:
                    for n8 in range(8 - NSC, 8):
                        scatter(u - 1, 1 - g, n8).wait()
                stage(u + 1, 1 - g)
        for n8 in range(8 - NSC, 8):
            scatter(NU - 1, (NU - 1) % 2, n8).wait()
            scatter(NU - 2, (NU - 2) % 2, n8).wait()

    return sc_bias


_sc_bias = _make_sc_kernel()


@jax.jit
def kernel(T, table):
    del T
    ttp = jnp.pad(jnp.transpose(table), ((0, 0), (LPAD, 129)))
    l4 = jnp.stack(
        [
            jnp.stack(
                [
                    ttp[:, LPAD - si - (8 * u + 1) : LPAD - si - (8 * u + 1) + LM]
                    for si in range(8)
                ],
                axis=1,
            )
            for u in range(NU)
        ],
        axis=0,
    )
    return _sc_bias(l4.reshape(NU * NUM_HEADS, 8, LM))


# speedup vs baseline: 1.9005x; 1.0533x over previous
import functools

import jax
import jax.numpy as jnp
from jax import lax
from jax.experimental import pallas as pl
from jax.experimental.pallas import tpu as pltpu
from jax.experimental.pallas import tpu_sc as plsc

MAX_LEN = 2048
NUM_HEADS = 16
RB = 16
NU = 8
LM = 4224
LPAD = 136
WIN = 2944
NSC = 4

_info = plsc.get_sparse_core_info()
_NC, _NS = _info.num_cores, _info.num_subcores
_NW = _NC * _NS
_ROWS_PER = (NUM_HEADS * MAX_LEN) // _NW
_WPH = MAX_LEN // _ROWS_PER


def _make_sc_kernel():
    mesh = plsc.VectorSubcoreMesh(core_axis_name="c", subcore_axis_name="s")

    @functools.partial(
        pl.kernel,
        mesh=mesh,
        out_type=jax.ShapeDtypeStruct((NUM_HEADS, MAX_LEN, MAX_LEN), jnp.float32),
        scratch_types=[pltpu.VMEM((RB, WIN), jnp.float32)] * 2
        + [pltpu.SemaphoreType.DMA] * (2 * NSC + 2),
    )
    def sc_bias(l3_hbm, out_hbm, *scratch):
        win = scratch[:2]
        sems = scratch[2 : 2 + 2 * NSC]
        ssem = scratch[2 + 2 * NSC :]
        wid = lax.axis_index("s") * _NC + lax.axis_index("c")
        h = wid // _WPH
        p = wid % _WPH

        mbase = pl.multiple_of(1152 - 1024 * p, 128)

        def stage_copy(u, g):
            return pltpu.make_async_copy(
                l3_hbm.at[u * NUM_HEADS + h, :, pl.ds(mbase, WIN)], win[g], ssem[g]
            )

        def scatter(u, g, n8):
            row0 = pl.multiple_of(RB * (u + 64 * p + 8 * n8), 8)
            return pltpu.make_async_copy(
                win[g].at[:, pl.ds(128 * (7 - n8), MAX_LEN)],
                out_hbm.at[h, pl.ds(row0, RB), :],
                sems[NSC * g + (n8 % NSC)],
            )

        stage_copy(0, 0).start()
        for u in range(NU):
            g = u % 2
            stage_copy(u, g).wait()
            for n8 in range(NSC):
                scatter(u, g, n8).start()
            if u >= 1:
                for n8 in range(NSC, 8):
                    scatter(u - 1, 1 - g, n8).wait()
            if u + 1 < NU:
                stage_copy(u + 1, 1 - g).start()
            for n8 in range(NSC, 8):
                scatter(u, g, n8 - NSC).wait()
                scatter(u, g, n8).start()
        for n8 in range(NSC, 8):
            scatter(NU - 1, (NU - 1) % 2, n8).wait()

    return sc_bias


_sc_bias = _make_sc_kernel()


@jax.jit
def kernel(T, table):
    del T
    ttp = jnp.pad(jnp.transpose(table), ((0, 0), (LPAD, 129)))
    l4 = jnp.stack(
        [
            jnp.stack(
                [
                    ttp[:, LPAD - si - (RB * u + 1) : LPAD - si - (RB * u + 1) + LM]
                    for si in range(RB)
                ],
                axis=1,
            )
            for u in range(NU)
        ],
        axis=0,
    )
    return _sc_bias(l4.reshape(NU * NUM_HEADS, RB, LM))


# speedup vs baseline: 1.9282x; 1.0146x over previous
import functools

import jax
import jax.numpy as jnp
from jax import lax
from jax.experimental import pallas as pl
from jax.experimental.pallas import tpu as pltpu
from jax.experimental.pallas import tpu_sc as plsc

MAX_LEN = 2048
NUM_HEADS = 16
RB = 16
NU = 8
LM = 4224
LPAD = 136
WIN = 2944
NSC = 4

_info = plsc.get_sparse_core_info()
_NC, _NS = _info.num_cores, _info.num_subcores
_NW = _NC * _NS
_ROWS_PER = (NUM_HEADS * MAX_LEN) // _NW
_WPH = MAX_LEN // _ROWS_PER


def _make_sc_kernel():
    mesh = plsc.VectorSubcoreMesh(core_axis_name="c", subcore_axis_name="s")

    @functools.partial(
        pl.kernel,
        mesh=mesh,
        out_type=jax.ShapeDtypeStruct((NUM_HEADS, MAX_LEN, MAX_LEN), jnp.float32),
        scratch_types=[pltpu.VMEM((RB, WIN), jnp.float32)] * 2
        + [pltpu.SemaphoreType.DMA] * (2 * NSC + 2),
    )
    def sc_bias(l3_hbm, out_hbm, *scratch):
        win = scratch[:2]
        sems = scratch[2 : 2 + 2 * NSC]
        ssem = scratch[2 + 2 * NSC :]
        wid = lax.axis_index("s") * _NC + lax.axis_index("c")
        h = wid // _WPH
        p = wid % _WPH

        mbase = pl.multiple_of(1152 - 1024 * p, 128)

        def stage_copy(u, g):
            return pltpu.make_async_copy(
                l3_hbm.at[u * NUM_HEADS + h, :, pl.ds(mbase, WIN)], win[g], ssem[g]
            )

        def stage_wait(g):
            pltpu.make_async_copy(
                l3_hbm.at[0, :, pl.ds(0, WIN)], win[g], ssem[g]
            ).wait()

        def scatter(u, g, n8):
            row0 = pl.multiple_of(RB * (u + 64 * p + 8 * n8), 8)
            return pltpu.make_async_copy(
                win[g].at[:, pl.ds(128 * (7 - n8), MAX_LEN)],
                out_hbm.at[h, pl.ds(row0, RB), :],
                sems[NSC * g + (n8 % NSC)],
            )

        def scatter_wait(g, slot):
            pltpu.make_async_copy(
                win[g].at[:, pl.ds(0, MAX_LEN)],
                out_hbm.at[h, pl.ds(0, RB), :],
                sems[NSC * g + slot],
            ).wait()

        stage_copy(0, 0).start()

        def blk(u2, carry):
            for g in range(2):
                u = 2 * u2 + g
                stage_wait(g)
                for n8 in range(NSC):
                    scatter(u, g, n8).start()
                if g == 1:
                    for n8 in range(NSC, 8):
                        scatter_wait(1 - g, n8 % NSC)
                    @pl.when(u2 < NU // 2 - 1)
                    def _stage_next():
                        stage_copy(u + 1, 1 - g).start()
                else:
                    @pl.when(u2 > 0)
                    def _drain_prev():
                        for n8 in range(NSC, 8):
                            scatter_wait(1 - g, n8 % NSC)

                    stage_copy(u + 1, 1 - g).start()

                for n8 in range(NSC, 8):
                    scatter_wait(g, n8 % NSC)
                    scatter(u, g, n8).start()
            return carry

        lax.fori_loop(0, NU // 2, blk, 0)
        for n8 in range(NSC, 8):
            scatter_wait(1, n8 % NSC)

    return sc_bias


_sc_bias = _make_sc_kernel()


@jax.jit
def kernel(T, table):
    del T
    ttp = jnp.pad(jnp.transpose(table), ((0, 0), (LPAD, 129)))
    l4 = jnp.stack(
        [
            jnp.stack(
                [
                    ttp[:, LPAD - si - (RB * u + 1) : LPAD - si - (RB * u + 1) + LM]
                    for si in range(RB)
                ],
                axis=1,
            )
            for u in range(NU)
        ],
        axis=0,
    )
    return _sc_bias(l4.reshape(NU * NUM_HEADS, RB, LM))
